# Initial kernel scaffold; baseline (speedup 1.0000x reference)
import jax
import jax.numpy as jnp
from jax.experimental import pallas as pl


def kernel(x, edge_index, W0, b0, W1, b1, Wout, bout):
    raise NotImplementedError("write your pallas kernel here")


# speedup vs baseline: 10.8047x; 10.8047x over previous
import functools

import jax
import jax.numpy as jnp
from jax import lax
from jax.experimental import pallas as pl
from jax.experimental.pallas import tpu as pltpu
from jax.experimental.pallas import tpu_sc as plsc

N = 10000
D = 128
NCORE = 2
NSUB = 16
NP = 10112
ROWS_PER_TILE = NP // NSUB
CH = 128
DEGW = 16
RB = 400
GRID = N // RB


def _sc_mesh():
    return plsc.VectorSubcoreMesh(core_axis_name="c", subcore_axis_name="s")


def _make_deg(E):
    ept = E // (NCORE * NSUB)
    nfull, rem = divmod(ept, CH)
    assert rem % 8 == 0

    @functools.partial(
        pl.kernel,
        mesh=_sc_mesh(),
        out_type=jax.ShapeDtypeStruct((NCORE, NP, D), jnp.float32),
        scratch_types=[
            pltpu.VMEM((CH,), jnp.int32),
            pltpu.VMEM((max(rem, 8),), jnp.int32),
            pltpu.VMEM((CH, D), jnp.float32),
            pltpu.VMEM_SHARED((NP, D), jnp.float32),
        ],
    )
    def deg_kernel(dst_hbm, ones_hbm, zrows_hbm, out_hbm, idx_v, idx_r, ones_v, acc_sh):
        c = lax.axis_index("c")
        s = lax.axis_index("s")
        pltpu.sync_copy(ones_hbm, ones_v)
        row0 = s * ROWS_PER_TILE
        pltpu.sync_copy(zrows_hbm, acc_sh.at[pl.ds(row0, ROWS_PER_TILE)])
        plsc.subcore_barrier()

        ebase = c * (E // NCORE) + s * ept

        def body(i, carry):
            pltpu.sync_copy(dst_hbm.at[pl.ds(ebase + i * CH, CH)], idx_v)
            pltpu.sync_copy(ones_v, acc_sh.at[idx_v], add=True)
            return carry

        lax.fori_loop(0, nfull, body, 0)
        if rem:
            pltpu.sync_copy(dst_hbm.at[pl.ds(ebase + nfull * CH, rem)], idx_r)
            pltpu.sync_copy(ones_v.at[pl.ds(0, rem)], acc_sh.at[idx_r], add=True)
        plsc.subcore_barrier()
        pltpu.sync_copy(
            acc_sh.at[pl.ds(row0, ROWS_PER_TILE)],
            out_hbm.at[c, pl.ds(row0, ROWS_PER_TILE)],
        )

    return deg_kernel


def _make_agg(E):
    CHA = 64
    ept = E // (NCORE * NSUB)
    nfull, rem = divmod(ept, CHA)
    assert rem % 8 == 0

    @functools.partial(
        pl.kernel,
        mesh=_sc_mesh(),
        out_type=jax.ShapeDtypeStruct((NCORE, NP, D), jnp.float32),
        scratch_types=[
            pltpu.VMEM((CHA,), jnp.int32),
            pltpu.VMEM((CHA,), jnp.int32),
            pltpu.VMEM((max(rem, 8),), jnp.int32),
            pltpu.VMEM((max(rem, 8),), jnp.int32),
            pltpu.VMEM((CHA, D), jnp.float32),
            pltpu.VMEM_SHARED((NP, D), jnp.float32),
            pltpu.SemaphoreType.DMA,
        ],
    )
    def agg_kernel(src_hbm, dst_hbm, g_hbm, zrows_hbm, out_hbm,
                   si_v, di_v, si_r, di_r, rows_v, acc_sh, sem):
        c = lax.axis_index("c")
        s = lax.axis_index("s")

        row0 = s * ROWS_PER_TILE
        pltpu.sync_copy(zrows_hbm, acc_sh.at[pl.ds(row0, ROWS_PER_TILE)])
        plsc.subcore_barrier()

        ebase = c * (E // NCORE) + s * ept

        def body(i, carry):
            off = ebase + i * CHA
            pltpu.sync_copy(src_hbm.at[pl.ds(off, CHA)], si_v)
            pltpu.async_copy(g_hbm.at[si_v], rows_v, sem).wait()
            pltpu.sync_copy(dst_hbm.at[pl.ds(off, CHA)], di_v)
            pltpu.sync_copy(rows_v, acc_sh.at[di_v], add=True)
            return carry

        lax.fori_loop(0, nfull, body, 0)
        if rem:
            off = ebase + nfull * CHA
            pltpu.sync_copy(src_hbm.at[pl.ds(off, rem)], si_r)
            pltpu.async_copy(g_hbm.at[si_r], rows_v.at[pl.ds(0, rem)], sem).wait()
            pltpu.sync_copy(dst_hbm.at[pl.ds(off, rem)], di_r)
            pltpu.sync_copy(rows_v.at[pl.ds(0, rem)], acc_sh.at[di_r], add=True)
        plsc.subcore_barrier()
        pltpu.sync_copy(
            acc_sh.at[pl.ds(row0, ROWS_PER_TILE)],
            out_hbm.at[c, pl.ds(row0, ROWS_PER_TILE)],
        )

    return agg_kernel


def _make_gather_dbg(E):
    CHG = 64
    ept = E // (NCORE * NSUB)
    nfull, rem = divmod(ept, CHG)

    @functools.partial(
        pl.kernel,
        mesh=_sc_mesh(),
        out_type=jax.ShapeDtypeStruct((E, D), jnp.float32),
        scratch_types=[
            pltpu.VMEM((CHG,), jnp.int32),
            pltpu.VMEM((max(rem, 8),), jnp.int32),
            pltpu.VMEM((CHG, D), jnp.float32),
            pltpu.SemaphoreType.DMA,
        ],
    )
    def gather_kernel(src_hbm, g_hbm, out_hbm, si_v, si_r, rows_v, sem):
        c = lax.axis_index("c")
        s = lax.axis_index("s")
        ebase = c * (E // NCORE) + s * ept

        def body(i, carry):
            off = ebase + i * CHG
            pltpu.sync_copy(src_hbm.at[pl.ds(off, CHG)], si_v)
            pltpu.async_copy(g_hbm.at[si_v], rows_v, sem).wait()
            pltpu.sync_copy(rows_v, out_hbm.at[pl.ds(off, CHG)])
            return carry

        lax.fori_loop(0, nfull, body, 0)
        if rem:
            off = ebase + nfull * CHG
            pltpu.sync_copy(src_hbm.at[pl.ds(off, rem)], si_r)
            pltpu.async_copy(g_hbm.at[si_r], rows_v.at[pl.ds(0, rem)], sem).wait()
            pltpu.sync_copy(rows_v.at[pl.ds(0, rem)], out_hbm.at[pl.ds(off, rem)])

    return gather_kernel


def _tc1_body(x_ref, w_ref, d0_ref, d1_ref, g_ref, dinv_ref):
    deg = d0_ref[0, :, 0:1] + d1_ref[0, :, 0:1] + 1.0
    dinv = lax.rsqrt(deg)
    h = jnp.dot(x_ref[...], w_ref[...], preferred_element_type=jnp.float32)
    g_ref[...] = dinv * h
    dinv_ref[...] = dinv


def _tc1(x, W0, degp):
    return pl.pallas_call(
        _tc1_body,
        grid=(GRID,),
        in_specs=[
            pl.BlockSpec((RB, D), lambda i: (i, 0)),
            pl.BlockSpec((D, D), lambda i: (0, 0)),
            pl.BlockSpec((1, RB, D), lambda i: (0, i, 0)),
            pl.BlockSpec((1, RB, D), lambda i: (1, i, 0)),
        ],
        out_specs=[
            pl.BlockSpec((RB, D), lambda i: (i, 0)),
            pl.BlockSpec((RB, 1), lambda i: (i, 0)),
        ],
        out_shape=[
            jax.ShapeDtypeStruct((N, D), jnp.float32),
            jax.ShapeDtypeStruct((N, 1), jnp.float32),
        ],
    )(x, W0, degp, degp)


def _tc2_body(s0_ref, s1_ref, g_ref, dinv_ref, b_ref, w_ref, out_ref):
    z = dinv_ref[...] * (s0_ref[0] + s1_ref[0] + g_ref[...]) + b_ref[...]
    h = jnp.dot(z, w_ref[...], preferred_element_type=jnp.float32)
    out_ref[...] = dinv_ref[...] * h


def _tc2(s, g, dinv, b, W):
    return pl.pallas_call(
        _tc2_body,
        grid=(GRID,),
        in_specs=[
            pl.BlockSpec((1, RB, D), lambda i: (0, i, 0)),
            pl.BlockSpec((1, RB, D), lambda i: (1, i, 0)),
            pl.BlockSpec((RB, D), lambda i: (i, 0)),
            pl.BlockSpec((RB, 1), lambda i: (i, 0)),
            pl.BlockSpec((1, D), lambda i: (0, 0)),
            pl.BlockSpec((D, D), lambda i: (0, 0)),
        ],
        out_specs=pl.BlockSpec((RB, D), lambda i: (i, 0)),
        out_shape=jax.ShapeDtypeStruct((N, D), jnp.float32),
    )(s, s, g, dinv, b, W)


def _tc3_body(s0_ref, s1_ref, g_ref, dinv_ref, b_ref, w_ref, bo_ref, out_ref):
    z = dinv_ref[...] * (s0_ref[0] + s1_ref[0] + g_ref[...]) + b_ref[...]
    r = jnp.maximum(z, 0.0)
    logits = jnp.dot(r, w_ref[...], preferred_element_type=jnp.float32) + bo_ref[...]
    m = jnp.max(logits, axis=1, keepdims=True)
    lse = jnp.log(jnp.sum(jnp.exp(logits - m), axis=1, keepdims=True)) + m
    out_ref[...] = logits - lse


def _tc3(s, g, dinv, b, Wout, bout):
    nc = Wout.shape[1]
    return pl.pallas_call(
        _tc3_body,
        grid=(GRID,),
        in_specs=[
            pl.BlockSpec((1, RB, D), lambda i: (0, i, 0)),
            pl.BlockSpec((1, RB, D), lambda i: (1, i, 0)),
            pl.BlockSpec((RB, D), lambda i: (i, 0)),
            pl.BlockSpec((RB, 1), lambda i: (i, 0)),
            pl.BlockSpec((1, D), lambda i: (0, 0)),
            pl.BlockSpec((D, nc), lambda i: (0, 0)),
            pl.BlockSpec((1, nc), lambda i: (0, 0)),
        ],
        out_specs=pl.BlockSpec((RB, nc), lambda i: (i, 0)),
        out_shape=jax.ShapeDtypeStruct((N, nc), jnp.float32),
    )(s, s, g, dinv, b, Wout, bout)


def _agg_jnp(src, dst, g):
    E = src.shape[0]
    h = E // 2
    def part(s_, d_):
        return jnp.zeros((NP, D), jnp.float32).at[d_].add(g[s_])
    return jnp.stack([part(src[:h], dst[:h]), part(src[h:], dst[h:])])


def kernel(x, edge_index, W0, b0, W1, b1, Wout, bout):
    E = edge_index.shape[1]
    ei = edge_index.astype(jnp.int32)
    src, dst = ei[0], ei[1]
    zrows = jnp.zeros((ROWS_PER_TILE, D), jnp.float32)
    ones = jnp.zeros((CH, D), jnp.float32).at[:, 0].set(1.0)
    degp = _make_deg(E)(dst, ones, zrows)
    g0, dinv = _tc1(x, W0, degp)
    agg = _make_agg(E)
    s0 = agg(src, dst, g0, zrows)
    g1 = _tc2(s0, g0, dinv, b0.reshape(1, D), W1)
    s1 = agg(src, dst, g1, zrows)
    return _tc3(s1, g1, dinv, b1.reshape(1, D), Wout, bout.reshape(1, -1))


# speedup vs baseline: 19.1324x; 1.7708x over previous
import functools

import jax
import jax.numpy as jnp
from jax import lax
from jax.experimental import pallas as pl
from jax.experimental.pallas import tpu as pltpu
from jax.experimental.pallas import tpu_sc as plsc

N = 10000
D = 128
NCORE = 2
NSUB = 16
NP = 10112
ROWS_PER_TILE = NP // NSUB
CH = 128
DEGW = 16
RB = 400
GRID = N // RB


def _sc_mesh():
    return plsc.VectorSubcoreMesh(core_axis_name="c", subcore_axis_name="s")


def _make_deg(E):
    ept = E // (NCORE * NSUB)
    nfull, rem = divmod(ept, CH)
    assert rem % 8 == 0

    @functools.partial(
        pl.kernel,
        mesh=_sc_mesh(),
        out_type=jax.ShapeDtypeStruct((NCORE, NP, D), jnp.float32),
        scratch_types=[
            pltpu.VMEM((CH,), jnp.int32),
            pltpu.VMEM((max(rem, 8),), jnp.int32),
            pltpu.VMEM((CH, D), jnp.float32),
            pltpu.VMEM_SHARED((NP, D), jnp.float32),
        ],
    )
    def deg_kernel(dst_hbm, ones_hbm, zrows_hbm, out_hbm, idx_v, idx_r, ones_v, acc_sh):
        c = lax.axis_index("c")
        s = lax.axis_index("s")
        pltpu.sync_copy(ones_hbm, ones_v)
        row0 = s * ROWS_PER_TILE
        pltpu.sync_copy(zrows_hbm, acc_sh.at[pl.ds(row0, ROWS_PER_TILE)])
        plsc.subcore_barrier()

        ebase = c * (E // NCORE) + s * ept

        def body(i, carry):
            pltpu.sync_copy(dst_hbm.at[pl.ds(ebase + i * CH, CH)], idx_v)
            pltpu.sync_copy(ones_v, acc_sh.at[idx_v], add=True)
            return carry

        lax.fori_loop(0, nfull, body, 0)
        if rem:
            pltpu.sync_copy(dst_hbm.at[pl.ds(ebase + nfull * CH, rem)], idx_r)
            pltpu.sync_copy(ones_v.at[pl.ds(0, rem)], acc_sh.at[idx_r], add=True)
        plsc.subcore_barrier()
        pltpu.sync_copy(
            acc_sh.at[pl.ds(row0, ROWS_PER_TILE)],
            out_hbm.at[c, pl.ds(row0, ROWS_PER_TILE)],
        )

    return deg_kernel


def _make_agg(E):
    GG = 128
    HG = 64
    ept = E // (NCORE * NSUB)
    npairs, rem = divmod(ept, 2 * GG)
    assert rem == 16

    @functools.partial(
        pl.kernel,
        mesh=_sc_mesh(),
        out_type=jax.ShapeDtypeStruct((NCORE, NP, D), jnp.float32),
        scratch_types=[
            pltpu.VMEM((GG,), jnp.int32),
            pltpu.VMEM((GG,), jnp.int32),
            pltpu.VMEM((GG,), jnp.int32),
            pltpu.VMEM((GG,), jnp.int32),
            pltpu.VMEM((rem,), jnp.int32),
            pltpu.VMEM((rem,), jnp.int32),
            pltpu.VMEM((GG, D), jnp.float32),
            pltpu.VMEM((GG, D), jnp.float32),
            pltpu.VMEM_SHARED((NP, D), jnp.float32),
            pltpu.SemaphoreType.DMA,
            pltpu.SemaphoreType.DMA,
        ],
    )
    def agg_kernel(src_hbm, dst_hbm, g_hbm, zrows_hbm, out_hbm,
                   si_a, di_a, si_b, di_b, si_r, di_r,
                   rows_a, rows_b, acc_sh, sem_a, sem_b):
        c = lax.axis_index("c")
        s = lax.axis_index("s")

        row0 = s * ROWS_PER_TILE
        pltpu.sync_copy(zrows_hbm, acc_sh.at[pl.ds(row0, ROWS_PER_TILE)])
        plsc.subcore_barrier()

        ebase = c * (E // NCORE) + s * ept

        def group(off, si, di, rows, sem):
            pltpu.sync_copy(src_hbm.at[pl.ds(off, GG)], si)
            h1 = pltpu.async_copy(g_hbm.at[si.at[pl.ds(0, HG)]],
                                  rows.at[pl.ds(0, HG)], sem)
            h2 = pltpu.async_copy(g_hbm.at[si.at[pl.ds(HG, HG)]],
                                  rows.at[pl.ds(HG, HG)], sem)
            pltpu.sync_copy(dst_hbm.at[pl.ds(off, GG)], di)
            return h1, h2

        def drain(h1, h2, di, rows):
            h1.wait()
            h2.wait()
            pltpu.sync_copy(rows, acc_sh.at[di], add=True)

        def body(i, carry):
            off = ebase + i * (2 * GG)
            ha1, ha2 = group(off, si_a, di_a, rows_a, sem_a)
            hb1, hb2 = group(off + GG, si_b, di_b, rows_b, sem_b)
            drain(ha1, ha2, di_a, rows_a)
            drain(hb1, hb2, di_b, rows_b)
            return carry

        lax.fori_loop(0, npairs, body, 0)
        off = ebase + npairs * 2 * GG
        pltpu.sync_copy(src_hbm.at[pl.ds(off, rem)], si_r)
        pltpu.async_copy(g_hbm.at[si_r], rows_b.at[pl.ds(0, rem)], sem_b).wait()
        pltpu.sync_copy(dst_hbm.at[pl.ds(off, rem)], di_r)
        pltpu.sync_copy(rows_b.at[pl.ds(0, rem)], acc_sh.at[di_r], add=True)
        plsc.subcore_barrier()
        pltpu.sync_copy(
            acc_sh.at[pl.ds(row0, ROWS_PER_TILE)],
            out_hbm.at[c, pl.ds(row0, ROWS_PER_TILE)],
        )

    return agg_kernel


def _make_gather_dbg(E):
    CHG = 64
    ept = E // (NCORE * NSUB)
    nfull, rem = divmod(ept, CHG)

    @functools.partial(
        pl.kernel,
        mesh=_sc_mesh(),
        out_type=jax.ShapeDtypeStruct((E, D), jnp.float32),
        scratch_types=[
            pltpu.VMEM((CHG,), jnp.int32),
            pltpu.VMEM((max(rem, 8),), jnp.int32),
            pltpu.VMEM((CHG, D), jnp.float32),
            pltpu.SemaphoreType.DMA,
        ],
    )
    def gather_kernel(src_hbm, g_hbm, out_hbm, si_v, si_r, rows_v, sem):
        c = lax.axis_index("c")
        s = lax.axis_index("s")
        ebase = c * (E // NCORE) + s * ept

        def body(i, carry):
            off = ebase + i * CHG
            pltpu.sync_copy(src_hbm.at[pl.ds(off, CHG)], si_v)
            pltpu.async_copy(g_hbm.at[si_v], rows_v, sem).wait()
            pltpu.sync_copy(rows_v, out_hbm.at[pl.ds(off, CHG)])
            return carry

        lax.fori_loop(0, nfull, body, 0)
        if rem:
            off = ebase + nfull * CHG
            pltpu.sync_copy(src_hbm.at[pl.ds(off, rem)], si_r)
            pltpu.async_copy(g_hbm.at[si_r], rows_v.at[pl.ds(0, rem)], sem).wait()
            pltpu.sync_copy(rows_v.at[pl.ds(0, rem)], out_hbm.at[pl.ds(off, rem)])

    return gather_kernel


def _tc1_body(x_ref, w_ref, d0_ref, d1_ref, g_ref, dinv_ref):
    deg = d0_ref[0, :, 0:1] + d1_ref[0, :, 0:1] + 1.0
    dinv = lax.rsqrt(deg)
    h = jnp.dot(x_ref[...], w_ref[...], preferred_element_type=jnp.float32)
    g_ref[...] = dinv * h
    dinv_ref[...] = dinv


def _tc1(x, W0, degp):
    return pl.pallas_call(
        _tc1_body,
        grid=(GRID,),
        in_specs=[
            pl.BlockSpec((RB, D), lambda i: (i, 0)),
            pl.BlockSpec((D, D), lambda i: (0, 0)),
            pl.BlockSpec((1, RB, D), lambda i: (0, i, 0)),
            pl.BlockSpec((1, RB, D), lambda i: (1, i, 0)),
        ],
        out_specs=[
            pl.BlockSpec((RB, D), lambda i: (i, 0)),
            pl.BlockSpec((RB, 1), lambda i: (i, 0)),
        ],
        out_shape=[
            jax.ShapeDtypeStruct((N, D), jnp.float32),
            jax.ShapeDtypeStruct((N, 1), jnp.float32),
        ],
    )(x, W0, degp, degp)


def _tc2_body(s0_ref, s1_ref, g_ref, dinv_ref, b_ref, w_ref, out_ref):
    z = dinv_ref[...] * (s0_ref[0] + s1_ref[0] + g_ref[...]) + b_ref[...]
    h = jnp.dot(z, w_ref[...], preferred_element_type=jnp.float32)
    out_ref[...] = dinv_ref[...] * h


def _tc2(s, g, dinv, b, W):
    return pl.pallas_call(
        _tc2_body,
        grid=(GRID,),
        in_specs=[
            pl.BlockSpec((1, RB, D), lambda i: (0, i, 0)),
            pl.BlockSpec((1, RB, D), lambda i: (1, i, 0)),
            pl.BlockSpec((RB, D), lambda i: (i, 0)),
            pl.BlockSpec((RB, 1), lambda i: (i, 0)),
            pl.BlockSpec((1, D), lambda i: (0, 0)),
            pl.BlockSpec((D, D), lambda i: (0, 0)),
        ],
        out_specs=pl.BlockSpec((RB, D), lambda i: (i, 0)),
        out_shape=jax.ShapeDtypeStruct((N, D), jnp.float32),
    )(s, s, g, dinv, b, W)


def _tc3_body(s0_ref, s1_ref, g_ref, dinv_ref, b_ref, w_ref, bo_ref, out_ref):
    z = dinv_ref[...] * (s0_ref[0] + s1_ref[0] + g_ref[...]) + b_ref[...]
    r = jnp.maximum(z, 0.0)
    logits = jnp.dot(r, w_ref[...], preferred_element_type=jnp.float32) + bo_ref[...]
    m = jnp.max(logits, axis=1, keepdims=True)
    lse = jnp.log(jnp.sum(jnp.exp(logits - m), axis=1, keepdims=True)) + m
    out_ref[...] = logits - lse


def _tc3(s, g, dinv, b, Wout, bout):
    nc = Wout.shape[1]
    return pl.pallas_call(
        _tc3_body,
        grid=(GRID,),
        in_specs=[
            pl.BlockSpec((1, RB, D), lambda i: (0, i, 0)),
            pl.BlockSpec((1, RB, D), lambda i: (1, i, 0)),
            pl.BlockSpec((RB, D), lambda i: (i, 0)),
            pl.BlockSpec((RB, 1), lambda i: (i, 0)),
            pl.BlockSpec((1, D), lambda i: (0, 0)),
            pl.BlockSpec((D, nc), lambda i: (0, 0)),
            pl.BlockSpec((1, nc), lambda i: (0, 0)),
        ],
        out_specs=pl.BlockSpec((RB, nc), lambda i: (i, 0)),
        out_shape=jax.ShapeDtypeStruct((N, nc), jnp.float32),
    )(s, s, g, dinv, b, Wout, bout)


def _agg_jnp(src, dst, g):
    E = src.shape[0]
    h = E // 2
    def part(s_, d_):
        return jnp.zeros((NP, D), jnp.float32).at[d_].add(g[s_])
    return jnp.stack([part(src[:h], dst[:h]), part(src[h:], dst[h:])])


def kernel(x, edge_index, W0, b0, W1, b1, Wout, bout):
    E = edge_index.shape[1]
    ei = edge_index.astype(jnp.int32)
    src, dst = ei[0], ei[1]
    zrows = jnp.zeros((ROWS_PER_TILE, D), jnp.float32)
    ones = jnp.zeros((CH, D), jnp.float32).at[:, 0].set(1.0)
    degp = _make_deg(E)(dst, ones, zrows)
    g0, dinv = _tc1(x, W0, degp)
    agg = _make_agg(E)
    s0 = agg(src, dst, g0, zrows)
    g1 = _tc2(s0, g0, dinv, b0.reshape(1, D), W1)
    s1 = agg(src, dst, g1, zrows)
    return _tc3(s1, g1, dinv, b1.reshape(1, D), Wout, bout.reshape(1, -1))


# speedup vs baseline: 20.1673x; 1.0541x over previous
import functools

import jax
import jax.numpy as jnp
from jax import lax
from jax.experimental import pallas as pl
from jax.experimental.pallas import tpu as pltpu
from jax.experimental.pallas import tpu_sc as plsc

N = 10000
D = 128
NCORE = 2
NSUB = 16
NP = 10112
ROWS_PER_TILE = NP // NSUB
CH = 128
DEGW = 16
RB = 400
GRID = N // RB


def _sc_mesh():
    return plsc.VectorSubcoreMesh(core_axis_name="c", subcore_axis_name="s")


def _make_deg(E):
    GG = 128
    SUP = 4 * GG
    nsup_all = E // SUP
    base_sup, extra = divmod(nsup_all, NCORE * NSUB)

    @functools.partial(
        pl.kernel,
        mesh=_sc_mesh(),
        out_type=jax.ShapeDtypeStruct((NCORE, NP, D), jnp.float32),
        scratch_types=[
            pltpu.VMEM((4, GG), jnp.int32),
            pltpu.VMEM((GG, D), jnp.float32),
            pltpu.VMEM_SHARED((NP, D), jnp.float32),
        ],
    )
    def deg_kernel(dst2_hbm, ones_hbm, zrows_hbm, out_hbm, di_v, ones_v, acc_sh):
        c = lax.axis_index("c")
        s = lax.axis_index("s")
        wid = c * NSUB + s
        pltpu.sync_copy(ones_hbm, ones_v)
        row0 = s * ROWS_PER_TILE
        pltpu.sync_copy(zrows_hbm, acc_sh.at[pl.ds(row0, ROWS_PER_TILE)])
        plsc.subcore_barrier()

        supbase = wid * base_sup + jnp.minimum(wid, extra)
        nsup = base_sup + jnp.where(wid < extra, 1, 0)

        def body(i, carry):
            pltpu.sync_copy(dst2_hbm.at[pl.ds((supbase + i) * 4, 4)], di_v)
            for j in range(4):
                pltpu.sync_copy(ones_v, acc_sh.at[di_v.at[j]], add=True)
            return carry

        lax.fori_loop(0, nsup, body, 0)
        plsc.subcore_barrier()
        pltpu.sync_copy(
            acc_sh.at[pl.ds(row0, ROWS_PER_TILE)],
            out_hbm.at[c, pl.ds(row0, ROWS_PER_TILE)],
        )

    return deg_kernel


def _make_agg(E):
    GG = 128
    HG = 64
    SUP = 4 * GG
    nsup_all = E // SUP
    assert nsup_all * SUP == E
    base_sup, extra = divmod(nsup_all, NCORE * NSUB)

    @functools.partial(
        pl.kernel,
        mesh=_sc_mesh(),
        out_type=jax.ShapeDtypeStruct((NCORE, NP, D), jnp.float32),
        scratch_types=[
            pltpu.VMEM((SUP,), jnp.int32),
            pltpu.VMEM((4, GG), jnp.int32),
            pltpu.VMEM((GG, D), jnp.float32),
            pltpu.VMEM((GG, D), jnp.float32),
            pltpu.VMEM_SHARED((NP, D), jnp.float32),
            pltpu.SemaphoreType.DMA,
            pltpu.SemaphoreType.DMA,
        ],
    )
    def agg_kernel(src_hbm, dst2_hbm, g_hbm, zrows_hbm, out_hbm,
                   si_v, di_v, rows_a, rows_b, acc_sh, sem_a, sem_b):
        c = lax.axis_index("c")
        s = lax.axis_index("s")
        wid = c * NSUB + s

        row0 = s * ROWS_PER_TILE
        pltpu.sync_copy(zrows_hbm, acc_sh.at[pl.ds(row0, ROWS_PER_TILE)])
        plsc.subcore_barrier()

        supbase = wid * base_sup + jnp.minimum(wid, extra)
        nsup = base_sup + jnp.where(wid < extra, 1, 0)

        def fire(j, rows, sem):
            h1 = pltpu.async_copy(g_hbm.at[si_v.at[pl.ds(j * GG, HG)]],
                                  rows.at[pl.ds(0, HG)], sem)
            h2 = pltpu.async_copy(g_hbm.at[si_v.at[pl.ds(j * GG + HG, HG)]],
                                  rows.at[pl.ds(HG, HG)], sem)
            return h1, h2

        def drain(hs, j, rows):
            hs[0].wait()
            hs[1].wait()
            pltpu.sync_copy(rows, acc_sh.at[di_v.at[j]], add=True)

        def body(i, carry):
            sup = supbase + i
            pltpu.sync_copy(src_hbm.at[pl.ds(sup * SUP, SUP)], si_v)
            pltpu.sync_copy(dst2_hbm.at[pl.ds(sup * 4, 4)], di_v)
            ha = fire(0, rows_a, sem_a)
            hb = fire(1, rows_b, sem_b)
            drain(ha, 0, rows_a)
            ha = fire(2, rows_a, sem_a)
            drain(hb, 1, rows_b)
            hb = fire(3, rows_b, sem_b)
            drain(ha, 2, rows_a)
            drain(hb, 3, rows_b)
            return carry

        lax.fori_loop(0, nsup, body, 0)
        plsc.subcore_barrier()
        pltpu.sync_copy(
            acc_sh.at[pl.ds(row0, ROWS_PER_TILE)],
            out_hbm.at[c, pl.ds(row0, ROWS_PER_TILE)],
        )

    return agg_kernel


def _make_gather_dbg(E):
    CHG = 64
    ept = E // (NCORE * NSUB)
    nfull, rem = divmod(ept, CHG)

    @functools.partial(
        pl.kernel,
        mesh=_sc_mesh(),
        out_type=jax.ShapeDtypeStruct((E, D), jnp.float32),
        scratch_types=[
            pltpu.VMEM((CHG,), jnp.int32),
            pltpu.VMEM((max(rem, 8),), jnp.int32),
            pltpu.VMEM((CHG, D), jnp.float32),
            pltpu.SemaphoreType.DMA,
        ],
    )
    def gather_kernel(src_hbm, g_hbm, out_hbm, si_v, si_r, rows_v, sem):
        c = lax.axis_index("c")
        s = lax.axis_index("s")
        ebase = c * (E // NCORE) + s * ept

        def body(i, carry):
            off = ebase + i * CHG
            pltpu.sync_copy(src_hbm.at[pl.ds(off, CHG)], si_v)
            pltpu.async_copy(g_hbm.at[si_v], rows_v, sem).wait()
            pltpu.sync_copy(rows_v, out_hbm.at[pl.ds(off, CHG)])
            return carry

        lax.fori_loop(0, nfull, body, 0)
        if rem:
            off = ebase + nfull * CHG
            pltpu.sync_copy(src_hbm.at[pl.ds(off, rem)], si_r)
            pltpu.async_copy(g_hbm.at[si_r], rows_v.at[pl.ds(0, rem)], sem).wait()
            pltpu.sync_copy(rows_v.at[pl.ds(0, rem)], out_hbm.at[pl.ds(off, rem)])

    return gather_kernel


def _tc1_body(x_ref, w_ref, d0_ref, d1_ref, g_ref, dinv_ref):
    deg = d0_ref[0, :, 0:1] + d1_ref[0, :, 0:1] + 1.0
    dinv = lax.rsqrt(deg)
    h = jnp.dot(x_ref[...], w_ref[...], preferred_element_type=jnp.float32)
    g_ref[...] = dinv * h
    dinv_ref[...] = dinv


def _tc1(x, W0, degp):
    return pl.pallas_call(
        _tc1_body,
        grid=(GRID,),
        in_specs=[
            pl.BlockSpec((RB, D), lambda i: (i, 0)),
            pl.BlockSpec((D, D), lambda i: (0, 0)),
            pl.BlockSpec((1, RB, D), lambda i: (0, i, 0)),
            pl.BlockSpec((1, RB, D), lambda i: (1, i, 0)),
        ],
        out_specs=[
            pl.BlockSpec((RB, D), lambda i: (i, 0)),
            pl.BlockSpec((RB, 1), lambda i: (i, 0)),
        ],
        out_shape=[
            jax.ShapeDtypeStruct((N, D), jnp.float32),
            jax.ShapeDtypeStruct((N, 1), jnp.float32),
        ],
    )(x, W0, degp, degp)


def _tc2_body(s0_ref, s1_ref, g_ref, dinv_ref, b_ref, w_ref, out_ref):
    z = dinv_ref[...] * (s0_ref[0] + s1_ref[0] + g_ref[...]) + b_ref[...]
    h = jnp.dot(z, w_ref[...], preferred_element_type=jnp.float32)
    out_ref[...] = dinv_ref[...] * h


def _tc2(s, g, dinv, b, W):
    return pl.pallas_call(
        _tc2_body,
        grid=(GRID,),
        in_specs=[
            pl.BlockSpec((1, RB, D), lambda i: (0, i, 0)),
            pl.BlockSpec((1, RB, D), lambda i: (1, i, 0)),
            pl.BlockSpec((RB, D), lambda i: (i, 0)),
            pl.BlockSpec((RB, 1), lambda i: (i, 0)),
            pl.BlockSpec((1, D), lambda i: (0, 0)),
            pl.BlockSpec((D, D), lambda i: (0, 0)),
        ],
        out_specs=pl.BlockSpec((RB, D), lambda i: (i, 0)),
        out_shape=jax.ShapeDtypeStruct((N, D), jnp.float32),
    )(s, s, g, dinv, b, W)


def _tc3_body(s0_ref, s1_ref, g_ref, dinv_ref, b_ref, w_ref, bo_ref, out_ref):
    z = dinv_ref[...] * (s0_ref[0] + s1_ref[0] + g_ref[...]) + b_ref[...]
    r = jnp.maximum(z, 0.0)
    logits = jnp.dot(r, w_ref[...], preferred_element_type=jnp.float32) + bo_ref[...]
    m = jnp.max(logits, axis=1, keepdims=True)
    lse = jnp.log(jnp.sum(jnp.exp(logits - m), axis=1, keepdims=True)) + m
    out_ref[...] = logits - lse


def _tc3(s, g, dinv, b, Wout, bout):
    nc = Wout.shape[1]
    return pl.pallas_call(
        _tc3_body,
        grid=(GRID,),
        in_specs=[
            pl.BlockSpec((1, RB, D), lambda i: (0, i, 0)),
            pl.BlockSpec((1, RB, D), lambda i: (1, i, 0)),
            pl.BlockSpec((RB, D), lambda i: (i, 0)),
            pl.BlockSpec((RB, 1), lambda i: (i, 0)),
            pl.BlockSpec((1, D), lambda i: (0, 0)),
            pl.BlockSpec((D, nc), lambda i: (0, 0)),
            pl.BlockSpec((1, nc), lambda i: (0, 0)),
        ],
        out_specs=pl.BlockSpec((RB, nc), lambda i: (i, 0)),
        out_shape=jax.ShapeDtypeStruct((N, nc), jnp.float32),
    )(s, s, g, dinv, b, Wout, bout)


def _agg_jnp(src, dst, g):
    E = src.shape[0]
    h = E // 2
    def part(s_, d_):
        return jnp.zeros((NP, D), jnp.float32).at[d_].add(g[s_])
    return jnp.stack([part(src[:h], dst[:h]), part(src[h:], dst[h:])])


def kernel(x, edge_index, W0, b0, W1, b1, Wout, bout):
    E = edge_index.shape[1]
    ei = edge_index.astype(jnp.int32)
    src, dst = ei[0], ei[1]
    dst2 = dst.reshape(E // 128, 128)
    zrows = jnp.zeros((ROWS_PER_TILE, D), jnp.float32)
    ones = jnp.zeros((128, D), jnp.float32).at[:, 0].set(1.0)
    degp = _make_deg(E)(dst2, ones, zrows)
    g0, dinv = _tc1(x, W0, degp)
    agg = _make_agg(E)
    s0 = agg(src, dst2, g0, zrows)
    g1 = _tc2(s0, g0, dinv, b0.reshape(1, D), W1)
    s1 = agg(src, dst2, g1, zrows)
    return _tc3(s1, g1, dinv, b1.reshape(1, D), Wout, bout.reshape(1, -1))
